# baseline (speedup 1.0000x reference)
import functools

import jax
import jax.numpy as jnp
from jax import lax
from jax.experimental import pallas as pl
from jax.experimental.pallas import tpu as pltpu
from jax.experimental.pallas import tpu_sc as plsc

N_PAD = 10240
RPT = N_PAD // 16
ZR = 64
EK = 128
HID = 128
HALF = 64
OUT = 16

_MESH = dict(core_axis_name="c", subcore_axis_name="s", num_cores=2,
             num_subcores=16)


def _rsqrt16(v):
    bits = lax.bitcast_convert_type(v, jnp.int32)
    y = lax.bitcast_convert_type(0x5F3759DF - (bits >> 1), jnp.float32)
    for _ in range(4):
        y = y * (1.5 - 0.5 * v * y * y)
    return y


def _tc_front(x_pad, W1, b1, W2, b2):
    BLK = 256

    def body(x_ref, w1_ref, b1_ref, w2_ref, b2_ref, o_ref):
        h = jnp.dot(x_ref[...], w1_ref[...],
                    preferred_element_type=jnp.float32) + b1_ref[...]
        h = jnp.maximum(h, 0.0)
        h = jnp.dot(h, w2_ref[...],
                    preferred_element_type=jnp.float32) + b2_ref[...]
        o_ref[0] = h[:, :HALF]
        o_ref[1] = h[:, HALF:]

    out = pl.pallas_call(
        body,
        grid=(N_PAD // BLK,),
        in_specs=[
            pl.BlockSpec((BLK, HID), lambda i: (i, 0)),
            pl.BlockSpec((HID, HID), lambda i: (0, 0)),
            pl.BlockSpec((1, HID), lambda i: (0, 0)),
            pl.BlockSpec((HID, HID), lambda i: (0, 0)),
            pl.BlockSpec((1, HID), lambda i: (0, 0)),
        ],
        out_specs=pl.BlockSpec((2, BLK, HALF), lambda i: (0, i, 0)),
        out_shape=jax.ShapeDtypeStruct((2, N_PAD, HALF), jnp.float32),
    )(x_pad, W1, b1.reshape(1, HID), W2, b2.reshape(1, HID))
    return out.reshape(2 * N_PAD, HALF)


def _tc_cls(h0h1, Wc, bc):
    BLK = 256
    nblk = N_PAD // BLK

    def body(a_ref, b_ref, wc_ref, bc_ref, o_ref):
        h = jnp.concatenate([a_ref[...], b_ref[...]], axis=1)
        h = jnp.maximum(h, 0.0)
        o_ref[...] = jnp.dot(h, wc_ref[...],
                             preferred_element_type=jnp.float32) + bc_ref[...]

    return pl.pallas_call(
        body,
        grid=(nblk,),
        in_specs=[
            pl.BlockSpec((BLK, HALF), lambda i: (i, 0)),
            pl.BlockSpec((BLK, HALF), lambda i: (i + nblk, 0)),
            pl.BlockSpec((HID, OUT), lambda i: (0, 0)),
            pl.BlockSpec((1, OUT), lambda i: (0, 0)),
        ],
        out_specs=pl.BlockSpec((BLK, OUT), lambda i: (i, 0)),
        out_shape=jax.ShapeDtypeStruct((N_PAD, OUT), jnp.float32),
    )(h0h1, h0h1, Wc, bc.reshape(1, OUT))


def _sc_prop_body(cpt, h_hbm, ed_hbm, cv_hbm, out_hbm, dis_hbm,
                  Y, DINV2, ebuf, ebuf1, dbuf0, dbuf1, sbuf0, sbuf1,
                  rows, rows1,
                  segb, zbuf, obuf, djbuf, dsbuf, cv_v, gsem0, gsem1,
                  ssem0, ssem1, esem0, esem1):
    c = lax.axis_index("c")
    s = lax.axis_index("s")
    rbase = s * RPT
    obase = c * N_PAD + s * RPT
    cbase = s * cpt
    nseg = RPT // ZR

    zero16 = jnp.zeros((16,), jnp.float32)
    one16 = jnp.ones((16,), jnp.float32)

    def _zb(i, carry):
        for f in range(HALF // 16):
            zbuf[i, pl.ds(16 * f, 16)] = zero16
        return carry
    lax.fori_loop(0, ZR, _zb, 0)

    def _zo(i, carry):
        obuf[i] = zero16
        return carry
    lax.fori_loop(0, EK, _zo, 0)

    pltpu.sync_copy(cv_hbm, cv_v)
    T = cv_v[...][0]

    def _z0(j, carry):
        pltpu.sync_copy(obuf, DINV2.at[pl.ds(rbase + EK * j, EK)])
        return carry
    lax.fori_loop(0, RPT // EK, _z0, 0)

    def _ob(i, carry):
        obuf[i] = one16
        return carry
    lax.fori_loop(0, EK, _ob, 0)
    plsc.subcore_barrier()

    def _dg(ch, carry):
        pltpu.sync_copy(ed_hbm.at[cbase + ch], ebuf)
        pltpu.sync_copy(obuf, DINV2.at[ebuf.at[1]], add=True)
        return carry
    lax.fori_loop(0, cpt, _dg, 0)
    plsc.subcore_barrier()

    def _dv(j, carry):
        pltpu.sync_copy(DINV2.at[pl.ds(rbase + ZR * j, ZR)], djbuf)

        def _d1(i, cc):
            djbuf[i] = 1.0 / jnp.maximum(djbuf[i], 1.0)
            return cc
        lax.fori_loop(0, ZR, _d1, 0)
        pltpu.sync_copy(djbuf, DINV2.at[pl.ds(rbase + ZR * j, ZR)])
        return carry
    lax.fori_loop(0, nseg, _dv, 0)

    @pl.when(c == 0)
    def _():
        def _dout(j, carry):
            pltpu.sync_copy(DINV2.at[pl.ds(rbase + ZR * j, ZR)], djbuf)

            def _d2(i, cc):
                dv = djbuf[i]
                dsbuf[i] = dv * _rsqrt16(dv)
                return cc
            lax.fori_loop(0, ZR, _d2, 0)
            pltpu.sync_copy(dsbuf, dis_hbm.at[pl.ds(rbase + ZR * j, ZR)])
            return carry
        lax.fori_loop(0, nseg, _dout, 0)

    def _init_seg(j, carry):
        pltpu.sync_copy(h_hbm.at[pl.ds(obase + ZR * j, ZR)], segb)
        pltpu.sync_copy(DINV2.at[pl.ds(rbase + ZR * j, ZR)], djbuf)

        def _s0(i, cc):
            dv = djbuf[i]
            sp = dv * _rsqrt16(dv)
            for f in range(HALF // 16):
                segb[i, pl.ds(16 * f, 16)] = segb[i, pl.ds(16 * f, 16)] * sp
            return cc
        lax.fori_loop(0, ZR, _s0, 0)
        pltpu.sync_copy(segb, out_hbm.at[pl.ds(obase + ZR * j, ZR)])
        pltpu.sync_copy(zbuf, Y.at[pl.ds(rbase + ZR * j, ZR)])
        return carry
    lax.fori_loop(0, nseg, _init_seg, 0)
    plsc.subcore_barrier()

    ebufs = [ebuf, ebuf1]
    dbufs = [dbuf0, dbuf1]
    sbufs = [sbuf0, sbuf1]
    rowss = [rows, rows1]
    gsems = [gsem0, gsem1]
    ssems = [ssem0, ssem1]
    esems = [esem0, esem1]
    npair = cpt // 2

    def _round(r, carry):
        for b in range(2):
            pltpu.async_copy(ed_hbm.at[cbase + b], ebufs[b], esems[b])

        def _pair(g, cc):
            hs = []
            for b in range(2):
                pltpu.make_async_copy(
                    ed_hbm.at[0], ebufs[b], esems[b]).wait()

                @pl.when(g > 0)
                def _(b=b):
                    pltpu.make_async_copy(
                        h_hbm.at[pl.ds(0, EK)], rowss[b], ssems[b]).wait()
                for q in range(EK // 16):
                    sbufs[b][pl.ds(16 * q, 16)] = (
                        ebufs[b][0, pl.ds(16 * q, 16)] + c * N_PAD)
                hs.append(pltpu.async_copy(
                    out_hbm.at[sbufs[b]], rowss[b], gsems[b]))
            for b in range(2):
                hs[b].wait()
                for q in range(EK // 16):
                    dbufs[b][pl.ds(16 * q, 16)] = ebufs[b][1,
                                                           pl.ds(16 * q, 16)]
                pltpu.async_copy(rowss[b], Y.at[dbufs[b]], ssems[b],
                                 add=True)

                @pl.when(g < npair - 1)
                def _(b=b, g=g):
                    pltpu.async_copy(ed_hbm.at[cbase + 2 * (g + 1) + b],
                                     ebufs[b], esems[b])
            return cc
        lax.fori_loop(0, npair, _pair, 0)
        for b in range(2):
            pltpu.make_async_copy(
                h_hbm.at[pl.ds(0, EK)], rowss[b], ssems[b]).wait()
        plsc.subcore_barrier()

        def _seg(j, cc):
            pltpu.sync_copy(Y.at[pl.ds(rbase + ZR * j, ZR)], segb)
            pltpu.sync_copy(DINV2.at[pl.ds(rbase + ZR * j, ZR)], djbuf)

            def _sr(i, c2):
                sp = djbuf[i]
                for f in range(HALF // 16):
                    segb[i, pl.ds(16 * f, 16)] = (
                        segb[i, pl.ds(16 * f, 16)] * sp)
                return c2
            lax.fori_loop(0, ZR, _sr, 0)
            pltpu.sync_copy(segb, out_hbm.at[pl.ds(obase + ZR * j, ZR)])
            pltpu.sync_copy(zbuf, Y.at[pl.ds(rbase + ZR * j, ZR)])
            return cc
        lax.fori_loop(0, nseg, _seg, 0)
        plsc.subcore_barrier()
        return carry
    lax.fori_loop(0, T, _round, 0)

    def _out_seg(j, carry):
        pltpu.sync_copy(out_hbm.at[pl.ds(obase + ZR * j, ZR)], segb)
        pltpu.sync_copy(DINV2.at[pl.ds(rbase + ZR * j, ZR)], djbuf)

        def _so(i, cc):
            sp = _rsqrt16(djbuf[i])
            for f in range(HALF // 16):
                segb[i, pl.ds(16 * f, 16)] = segb[i, pl.ds(16 * f, 16)] * sp
            return cc
        lax.fori_loop(0, ZR, _so, 0)
        pltpu.sync_copy(segb, out_hbm.at[pl.ds(obase + ZR * j, ZR)])
        return carry
    lax.fori_loop(0, nseg, _out_seg, 0)


def _sc_prop(h2, edges, cv, nch):
    cpt = nch // 16
    f = pl.kernel(
        functools.partial(_sc_prop_body, cpt),
        out_type=[jax.ShapeDtypeStruct((2 * N_PAD, HALF), jnp.float32),
                  jax.ShapeDtypeStruct((N_PAD, 16), jnp.float32)],
        mesh=plsc.VectorSubcoreMesh(**_MESH),
        compiler_params=pltpu.CompilerParams(use_tc_tiling_on_sc=False),
        scratch_types=[
            pltpu.VMEM_SHARED((N_PAD, HALF), jnp.float32),
            pltpu.VMEM_SHARED((N_PAD, 16), jnp.float32),
            pltpu.VMEM((2, EK), jnp.int32),
            pltpu.VMEM((2, EK), jnp.int32),
            pltpu.VMEM((EK,), jnp.int32),
            pltpu.VMEM((EK,), jnp.int32),
            pltpu.VMEM((EK,), jnp.int32),
            pltpu.VMEM((EK,), jnp.int32),
            pltpu.VMEM((EK, HALF), jnp.float32),
            pltpu.VMEM((EK, HALF), jnp.float32),
            pltpu.VMEM((ZR, HALF), jnp.float32),
            pltpu.VMEM((ZR, HALF), jnp.float32),
            pltpu.VMEM((EK, 16), jnp.float32),
            pltpu.VMEM((ZR, 16), jnp.float32),
            pltpu.VMEM((ZR, 16), jnp.float32),
            pltpu.VMEM((16,), jnp.int32),
            pltpu.SemaphoreType.DMA,
            pltpu.SemaphoreType.DMA,
            pltpu.SemaphoreType.DMA,
            pltpu.SemaphoreType.DMA,
            pltpu.SemaphoreType.DMA,
            pltpu.SemaphoreType.DMA,
        ],
    )
    return f(h2, edges, cv)


def _sc_final_body(cpt, h_hbm, ed_hbm, dis_hbm, y_hbm,
                   X16, Y16, ebuf, rows16, rowbuf16, zb16, dis2b):
    c = lax.axis_index("c")
    s = lax.axis_index("s")
    rbase = s * RPT
    cbase = s * cpt

    @pl.when(c == 0)
    def _():
        zero16 = jnp.zeros((16,), jnp.float32)

        def _zb(i, carry):
            zb16[i] = zero16
            return carry
        lax.fori_loop(0, ZR, _zb, 0)

        pltpu.sync_copy(dis_hbm.at[pl.ds(rbase, RPT)], dis2b)
        pltpu.sync_copy(h_hbm.at[pl.ds(rbase, RPT)], rowbuf16)

        def _s0(i, carry):
            rowbuf16[i] = rowbuf16[i] * dis2b[i]
            return carry
        lax.fori_loop(0, RPT, _s0, 0)
        pltpu.sync_copy(rowbuf16, X16.at[pl.ds(rbase, RPT)])

        def _zy(j, carry):
            pltpu.sync_copy(zb16, Y16.at[pl.ds(rbase + ZR * j, ZR)])
            return carry
        lax.fori_loop(0, RPT // ZR, _zy, 0)
        plsc.subcore_barrier()

        def _edge(ch, carry):
            pltpu.sync_copy(ed_hbm.at[cbase + ch], ebuf)
            pltpu.sync_copy(X16.at[ebuf.at[0]], rows16)
            pltpu.sync_copy(rows16, Y16.at[ebuf.at[1]], add=True)
            return carry
        lax.fori_loop(0, cpt, _edge, 0)
        plsc.subcore_barrier()

        pltpu.sync_copy(Y16.at[pl.ds(rbase, RPT)], rowbuf16)

        def _s1(i, carry):
            rowbuf16[i] = rowbuf16[i] * dis2b[i]
            return carry
        lax.fori_loop(0, RPT, _s1, 0)
        pltpu.sync_copy(rowbuf16, y_hbm.at[pl.ds(rbase, RPT)])


def _sc_final(h3, edges, dis, nch):
    cpt = nch // 16
    f = pl.kernel(
        functools.partial(_sc_final_body, cpt),
        out_type=jax.ShapeDtypeStruct((N_PAD, OUT), jnp.float32),
        mesh=plsc.VectorSubcoreMesh(**_MESH),
        compiler_params=pltpu.CompilerParams(use_tc_tiling_on_sc=False),
        scratch_types=[
            pltpu.VMEM_SHARED((N_PAD, OUT), jnp.float32),
            pltpu.VMEM_SHARED((N_PAD, OUT), jnp.float32),
            pltpu.VMEM((2, EK), jnp.int32),
            pltpu.VMEM((EK, OUT), jnp.float32),
            pltpu.VMEM((RPT, OUT), jnp.float32),
            pltpu.VMEM((ZR, OUT), jnp.float32),
            pltpu.VMEM((RPT, 16), jnp.float32),
        ],
    )
    return f(h3, edges, dis)


def kernel(x, edge_index, W1, b1, W2, b2, Wc, bc, conv_time):
    N = x.shape[0]
    E = edge_index.shape[1]
    ET = E + N
    nch = -(-ET // EK)
    nch = -(-nch // 16) * 16
    epad = nch * EK

    loop = jnp.arange(N, dtype=jnp.int32)
    padv = jnp.full((epad - ET,), N, jnp.int32)
    src = jnp.concatenate([edge_index[0].astype(jnp.int32), loop, padv])
    dst = jnp.concatenate([edge_index[1].astype(jnp.int32), loop, padv])
    edges = jnp.stack([src.reshape(nch, EK), dst.reshape(nch, EK)], axis=1)

    x_pad = jnp.pad(x, ((0, N_PAD - N), (0, 0)))
    cv = jnp.full((16,), conv_time, jnp.int32)

    h2 = _tc_front(x_pad, W1, b1, W2, b2)
    hp, dis = _sc_prop(h2, edges, cv, nch)
    h3 = _tc_cls(hp, Wc, bc)
    y = _sc_final(h3, edges, dis, nch)
    return y[:N]


# speedup vs baseline: 1.0803x; 1.0803x over previous
import functools

import jax
import jax.numpy as jnp
from jax import lax
from jax.experimental import pallas as pl
from jax.experimental.pallas import tpu as pltpu
from jax.experimental.pallas import tpu_sc as plsc

N_PAD = 10240
RPT = N_PAD // 16
ZR = 64
EK = 128
HID = 128
HALF = 64
OUT = 16

_MESH = dict(core_axis_name="c", subcore_axis_name="s", num_cores=2,
             num_subcores=16)


def _rsqrt16(v):
    bits = lax.bitcast_convert_type(v, jnp.int32)
    y = lax.bitcast_convert_type(0x5F3759DF - (bits >> 1), jnp.float32)
    for _ in range(4):
        y = y * (1.5 - 0.5 * v * y * y)
    return y


def _tc_front(x_pad, W1, b1, W2, b2):
    BLK = 256

    def body(x_ref, w1_ref, b1_ref, w2_ref, b2_ref, o_ref):
        h = jnp.dot(x_ref[...], w1_ref[...],
                    preferred_element_type=jnp.float32) + b1_ref[...]
        h = jnp.maximum(h, 0.0)
        h = jnp.dot(h, w2_ref[...],
                    preferred_element_type=jnp.float32) + b2_ref[...]
        o_ref[0] = h[:, :HALF]
        o_ref[1] = h[:, HALF:]

    out = pl.pallas_call(
        body,
        grid=(N_PAD // BLK,),
        in_specs=[
            pl.BlockSpec((BLK, HID), lambda i: (i, 0)),
            pl.BlockSpec((HID, HID), lambda i: (0, 0)),
            pl.BlockSpec((1, HID), lambda i: (0, 0)),
            pl.BlockSpec((HID, HID), lambda i: (0, 0)),
            pl.BlockSpec((1, HID), lambda i: (0, 0)),
        ],
        out_specs=pl.BlockSpec((2, BLK, HALF), lambda i: (0, i, 0)),
        out_shape=jax.ShapeDtypeStruct((2, N_PAD, HALF), jnp.float32),
    )(x_pad, W1, b1.reshape(1, HID), W2, b2.reshape(1, HID))
    return out.reshape(2 * N_PAD, HALF)


def _tc_cls(h0h1, Wc, bc):
    BLK = 256
    nblk = N_PAD // BLK

    def body(a_ref, b_ref, wc_ref, bc_ref, o_ref):
        h = jnp.concatenate([a_ref[...], b_ref[...]], axis=1)
        h = jnp.maximum(h, 0.0)
        o_ref[...] = jnp.dot(h, wc_ref[...],
                             preferred_element_type=jnp.float32) + bc_ref[...]

    return pl.pallas_call(
        body,
        grid=(nblk,),
        in_specs=[
            pl.BlockSpec((BLK, HALF), lambda i: (i, 0)),
            pl.BlockSpec((BLK, HALF), lambda i: (i + nblk, 0)),
            pl.BlockSpec((HID, OUT), lambda i: (0, 0)),
            pl.BlockSpec((1, OUT), lambda i: (0, 0)),
        ],
        out_specs=pl.BlockSpec((BLK, OUT), lambda i: (i, 0)),
        out_shape=jax.ShapeDtypeStruct((N_PAD, OUT), jnp.float32),
    )(h0h1, h0h1, Wc, bc.reshape(1, OUT))


def _sc_prop_body(cpt, h_hbm, ed_hbm, cv_hbm, out_hbm, dis_hbm,
                  X, Y, DINV2, ebuf, ebuf1, ebuf2, dbuf0, dbuf1, dbuf2,
                  rows, rows1, rows2,
                  segb, zbuf, obuf, djbuf, dsbuf, cv_v, gsem0, gsem1, gsem2,
                  ssem0, ssem1, ssem2, esem0, esem1, esem2):
    c = lax.axis_index("c")
    s = lax.axis_index("s")
    rbase = s * RPT
    obase = c * N_PAD + s * RPT
    cbase = s * cpt
    nseg = RPT // ZR

    zero16 = jnp.zeros((16,), jnp.float32)
    one16 = jnp.ones((16,), jnp.float32)

    def _zb(i, carry):
        for f in range(HALF // 16):
            zbuf[i, pl.ds(16 * f, 16)] = zero16
        return carry
    lax.fori_loop(0, ZR, _zb, 0)

    def _zo(i, carry):
        obuf[i] = zero16
        return carry
    lax.fori_loop(0, EK, _zo, 0)

    pltpu.sync_copy(cv_hbm, cv_v)
    T = cv_v[...][0]

    def _z0(j, carry):
        pltpu.sync_copy(obuf, DINV2.at[pl.ds(rbase + EK * j, EK)])
        return carry
    lax.fori_loop(0, RPT // EK, _z0, 0)

    def _ob(i, carry):
        obuf[i] = one16
        return carry
    lax.fori_loop(0, EK, _ob, 0)
    plsc.subcore_barrier()

    def _dg(ch, carry):
        pltpu.sync_copy(ed_hbm.at[cbase + ch], ebuf)
        pltpu.sync_copy(obuf, DINV2.at[ebuf.at[1]], add=True)
        return carry
    lax.fori_loop(0, cpt, _dg, 0)
    plsc.subcore_barrier()

    def _dv(j, carry):
        pltpu.sync_copy(DINV2.at[pl.ds(rbase + ZR * j, ZR)], djbuf)

        def _d1(i, cc):
            djbuf[i] = 1.0 / jnp.maximum(djbuf[i], 1.0)
            return cc
        lax.fori_loop(0, ZR, _d1, 0)
        pltpu.sync_copy(djbuf, DINV2.at[pl.ds(rbase + ZR * j, ZR)])
        return carry
    lax.fori_loop(0, nseg, _dv, 0)

    @pl.when(c == 0)
    def _():
        def _dout(j, carry):
            pltpu.sync_copy(DINV2.at[pl.ds(rbase + ZR * j, ZR)], djbuf)

            def _d2(i, cc):
                dv = djbuf[i]
                dsbuf[i] = dv * _rsqrt16(dv)
                return cc
            lax.fori_loop(0, ZR, _d2, 0)
            pltpu.sync_copy(dsbuf, dis_hbm.at[pl.ds(rbase + ZR * j, ZR)])
            return carry
        lax.fori_loop(0, nseg, _dout, 0)

    def _init_seg(j, carry):
        pltpu.sync_copy(h_hbm.at[pl.ds(obase + ZR * j, ZR)], segb)
        pltpu.sync_copy(DINV2.at[pl.ds(rbase + ZR * j, ZR)], djbuf)

        def _s0(i, cc):
            dv = djbuf[i]
            sp = dv * _rsqrt16(dv)
            for f in range(HALF // 16):
                segb[i, pl.ds(16 * f, 16)] = segb[i, pl.ds(16 * f, 16)] * sp
            return cc
        lax.fori_loop(0, ZR, _s0, 0)
        pltpu.sync_copy(segb, X.at[pl.ds(rbase + ZR * j, ZR)])
        pltpu.sync_copy(zbuf, Y.at[pl.ds(rbase + ZR * j, ZR)])
        return carry
    lax.fori_loop(0, nseg, _init_seg, 0)
    plsc.subcore_barrier()

    NB = 3
    ebufs = [ebuf, ebuf1, ebuf2]
    dbufs = [dbuf0, dbuf1, dbuf2]
    rowss = [rows, rows1, rows2]
    gsems = [gsem0, gsem1, gsem2]
    ssems = [ssem0, ssem1, ssem2]
    esems = [esem0, esem1, esem2]
    ngrp = cpt // NB

    def _round(r, carry):
        for b in range(NB):
            pltpu.async_copy(ed_hbm.at[cbase + b], ebufs[b], esems[b])

        def _grp(g, cc):
            hs = []
            for b in range(NB):
                pltpu.make_async_copy(
                    ed_hbm.at[0], ebufs[b], esems[b]).wait()

                @pl.when(g > 0)
                def _(b=b):
                    pltpu.make_async_copy(
                        h_hbm.at[pl.ds(0, EK)], rowss[b], ssems[b]).wait()
                hs.append(pltpu.async_copy(
                    X.at[ebufs[b].at[0]], rowss[b], gsems[b]))
            for b in range(NB):
                hs[b].wait()
                for q in range(EK // 16):
                    dbufs[b][pl.ds(16 * q, 16)] = ebufs[b][1,
                                                           pl.ds(16 * q, 16)]
                pltpu.async_copy(rowss[b], Y.at[dbufs[b]], ssems[b],
                                 add=True)

                @pl.when(g < ngrp - 1)
                def _(b=b, g=g):
                    pltpu.async_copy(ed_hbm.at[cbase + NB * (g + 1) + b],
                                     ebufs[b], esems[b])
            return cc
        lax.fori_loop(0, ngrp, _grp, 0)
        for b in range(NB):
            pltpu.make_async_copy(
                h_hbm.at[pl.ds(0, EK)], rowss[b], ssems[b]).wait()
        plsc.subcore_barrier()

        def _seg(j, cc):
            pltpu.sync_copy(Y.at[pl.ds(rbase + ZR * j, ZR)], segb)
            pltpu.sync_copy(DINV2.at[pl.ds(rbase + ZR * j, ZR)], djbuf)

            def _sr(i, c2):
                sp = djbuf[i]
                for f in range(HALF // 16):
                    segb[i, pl.ds(16 * f, 16)] = (
                        segb[i, pl.ds(16 * f, 16)] * sp)
                return c2
            lax.fori_loop(0, ZR, _sr, 0)
            pltpu.sync_copy(segb, X.at[pl.ds(rbase + ZR * j, ZR)])
            pltpu.sync_copy(zbuf, Y.at[pl.ds(rbase + ZR * j, ZR)])
            return cc
        lax.fori_loop(0, nseg, _seg, 0)
        plsc.subcore_barrier()
        return carry
    lax.fori_loop(0, T, _round, 0)

    def _out_seg(j, carry):
        pltpu.sync_copy(X.at[pl.ds(rbase + ZR * j, ZR)], segb)
        pltpu.sync_copy(DINV2.at[pl.ds(rbase + ZR * j, ZR)], djbuf)

        def _so(i, cc):
            sp = _rsqrt16(djbuf[i])
            for f in range(HALF // 16):
                segb[i, pl.ds(16 * f, 16)] = segb[i, pl.ds(16 * f, 16)] * sp
            return cc
        lax.fori_loop(0, ZR, _so, 0)
        pltpu.sync_copy(segb, out_hbm.at[pl.ds(obase + ZR * j, ZR)])
        return carry
    lax.fori_loop(0, nseg, _out_seg, 0)


def _sc_prop(h2, edges, cv, nch):
    cpt = nch // 16
    f = pl.kernel(
        functools.partial(_sc_prop_body, cpt),
        out_type=[jax.ShapeDtypeStruct((2 * N_PAD, HALF), jnp.float32),
                  jax.ShapeDtypeStruct((N_PAD, 16), jnp.float32)],
        mesh=plsc.VectorSubcoreMesh(**_MESH),
        compiler_params=pltpu.CompilerParams(use_tc_tiling_on_sc=False),
        scratch_types=[
            pltpu.VMEM_SHARED((N_PAD, HALF), jnp.float32),
            pltpu.VMEM_SHARED((N_PAD, HALF), jnp.float32),
            pltpu.VMEM_SHARED((N_PAD, 16), jnp.float32),
            pltpu.VMEM((2, EK), jnp.int32),
            pltpu.VMEM((2, EK), jnp.int32),
            pltpu.VMEM((2, EK), jnp.int32),
            pltpu.VMEM((EK,), jnp.int32),
            pltpu.VMEM((EK,), jnp.int32),
            pltpu.VMEM((EK,), jnp.int32),
            pltpu.VMEM((EK, HALF), jnp.float32),
            pltpu.VMEM((EK, HALF), jnp.float32),
            pltpu.VMEM((EK, HALF), jnp.float32),
            pltpu.VMEM((ZR, HALF), jnp.float32),
            pltpu.VMEM((ZR, HALF), jnp.float32),
            pltpu.VMEM((EK, 16), jnp.float32),
            pltpu.VMEM((ZR, 16), jnp.float32),
            pltpu.VMEM((ZR, 16), jnp.float32),
            pltpu.VMEM((16,), jnp.int32),
            pltpu.SemaphoreType.DMA,
            pltpu.SemaphoreType.DMA,
            pltpu.SemaphoreType.DMA,
            pltpu.SemaphoreType.DMA,
            pltpu.SemaphoreType.DMA,
            pltpu.SemaphoreType.DMA,
            pltpu.SemaphoreType.DMA,
            pltpu.SemaphoreType.DMA,
            pltpu.SemaphoreType.DMA,
        ],
    )
    return f(h2, edges, cv)


def _sc_final_body(cpt, h_hbm, ed_hbm, dis_hbm, y_hbm,
                   X16, Y16, ebuf, rows16, rowbuf16, zb16, dis2b):
    c = lax.axis_index("c")
    s = lax.axis_index("s")
    rbase = s * RPT
    cbase = s * cpt

    @pl.when(c == 0)
    def _():
        zero16 = jnp.zeros((16,), jnp.float32)

        def _zb(i, carry):
            zb16[i] = zero16
            return carry
        lax.fori_loop(0, ZR, _zb, 0)

        pltpu.sync_copy(dis_hbm.at[pl.ds(rbase, RPT)], dis2b)
        pltpu.sync_copy(h_hbm.at[pl.ds(rbase, RPT)], rowbuf16)

        def _s0(i, carry):
            rowbuf16[i] = rowbuf16[i] * dis2b[i]
            return carry
        lax.fori_loop(0, RPT, _s0, 0)
        pltpu.sync_copy(rowbuf16, X16.at[pl.ds(rbase, RPT)])

        def _zy(j, carry):
            pltpu.sync_copy(zb16, Y16.at[pl.ds(rbase + ZR * j, ZR)])
            return carry
        lax.fori_loop(0, RPT // ZR, _zy, 0)
        plsc.subcore_barrier()

        def _edge(ch, carry):
            pltpu.sync_copy(ed_hbm.at[cbase + ch], ebuf)
            pltpu.sync_copy(X16.at[ebuf.at[0]], rows16)
            pltpu.sync_copy(rows16, Y16.at[ebuf.at[1]], add=True)
            return carry
        lax.fori_loop(0, cpt, _edge, 0)
        plsc.subcore_barrier()

        pltpu.sync_copy(Y16.at[pl.ds(rbase, RPT)], rowbuf16)

        def _s1(i, carry):
            rowbuf16[i] = rowbuf16[i] * dis2b[i]
            return carry
        lax.fori_loop(0, RPT, _s1, 0)
        pltpu.sync_copy(rowbuf16, y_hbm.at[pl.ds(rbase, RPT)])


def _sc_final(h3, edges, dis, nch):
    cpt = nch // 16
    f = pl.kernel(
        functools.partial(_sc_final_body, cpt),
        out_type=jax.ShapeDtypeStruct((N_PAD, OUT), jnp.float32),
        mesh=plsc.VectorSubcoreMesh(**_MESH),
        compiler_params=pltpu.CompilerParams(use_tc_tiling_on_sc=False),
        scratch_types=[
            pltpu.VMEM_SHARED((N_PAD, OUT), jnp.float32),
            pltpu.VMEM_SHARED((N_PAD, OUT), jnp.float32),
            pltpu.VMEM((2, EK), jnp.int32),
            pltpu.VMEM((EK, OUT), jnp.float32),
            pltpu.VMEM((RPT, OUT), jnp.float32),
            pltpu.VMEM((ZR, OUT), jnp.float32),
            pltpu.VMEM((RPT, 16), jnp.float32),
        ],
    )
    return f(h3, edges, dis)


def kernel(x, edge_index, W1, b1, W2, b2, Wc, bc, conv_time):
    N = x.shape[0]
    E = edge_index.shape[1]
    ET = E + N
    nch = -(-ET // EK)
    nch = -(-nch // 16) * 16
    epad = nch * EK

    loop = jnp.arange(N, dtype=jnp.int32)
    padv = jnp.full((epad - ET,), N, jnp.int32)
    src = jnp.concatenate([edge_index[0].astype(jnp.int32), loop, padv])
    dst = jnp.concatenate([edge_index[1].astype(jnp.int32), loop, padv])
    edges = jnp.stack([src.reshape(nch, EK), dst.reshape(nch, EK)], axis=1)

    x_pad = jnp.pad(x, ((0, N_PAD - N), (0, 0)))
    cv = jnp.full((16,), conv_time, jnp.int32)

    h2 = _tc_front(x_pad, W1, b1, W2, b2)
    hp, dis = _sc_prop(h2, edges, cv, nch)
    h3 = _tc_cls(hp, Wc, bc)
    y = _sc_final(h3, edges, dis, nch)
    return y[:N]


# speedup vs baseline: 1.3839x; 1.2810x over previous
import functools

import jax
import jax.numpy as jnp
from jax import lax
from jax.experimental import pallas as pl
from jax.experimental.pallas import tpu as pltpu
from jax.experimental.pallas import tpu_sc as plsc

N_PAD = 10240
RPT = N_PAD // 16
ZR = 64
EK = 128
HID = 128
HALF = 64
OUT = 16

_MESH = dict(core_axis_name="c", subcore_axis_name="s", num_cores=2,
             num_subcores=16)


def _rsqrt16(v):
    bits = lax.bitcast_convert_type(v, jnp.int32)
    y = lax.bitcast_convert_type(0x5F3759DF - (bits >> 1), jnp.float32)
    for _ in range(4):
        y = y * (1.5 - 0.5 * v * y * y)
    return y


def _tc_front(x_pad, W1, b1, W2, b2):
    BLK = 256

    def body(x_ref, w1_ref, b1_ref, w2_ref, b2_ref, o_ref):
        h = jnp.dot(x_ref[...], w1_ref[...],
                    preferred_element_type=jnp.float32) + b1_ref[...]
        h = jnp.maximum(h, 0.0)
        h = jnp.dot(h, w2_ref[...],
                    preferred_element_type=jnp.float32) + b2_ref[...]
        o_ref[0] = h[:, :HALF]
        o_ref[1] = h[:, HALF:]

    out = pl.pallas_call(
        body,
        grid=(N_PAD // BLK,),
        in_specs=[
            pl.BlockSpec((BLK, HID), lambda i: (i, 0)),
            pl.BlockSpec((HID, HID), lambda i: (0, 0)),
            pl.BlockSpec((1, HID), lambda i: (0, 0)),
            pl.BlockSpec((HID, HID), lambda i: (0, 0)),
            pl.BlockSpec((1, HID), lambda i: (0, 0)),
        ],
        out_specs=pl.BlockSpec((2, BLK, HALF), lambda i: (0, i, 0)),
        out_shape=jax.ShapeDtypeStruct((2, N_PAD, HALF), jnp.float32),
    )(x_pad, W1, b1.reshape(1, HID), W2, b2.reshape(1, HID))
    return out.reshape(2 * N_PAD, HALF)


def _tc_cls(h0h1, Wc, bc):
    BLK = 256
    nblk = N_PAD // BLK

    def body(a_ref, b_ref, wc_ref, bc_ref, o_ref):
        h = jnp.concatenate([a_ref[...], b_ref[...]], axis=1)
        h = jnp.maximum(h, 0.0)
        o_ref[...] = jnp.dot(h, wc_ref[...],
                             preferred_element_type=jnp.float32) + bc_ref[...]

    return pl.pallas_call(
        body,
        grid=(nblk,),
        in_specs=[
            pl.BlockSpec((BLK, HALF), lambda i: (i, 0)),
            pl.BlockSpec((BLK, HALF), lambda i: (i + nblk, 0)),
            pl.BlockSpec((HID, OUT), lambda i: (0, 0)),
            pl.BlockSpec((1, OUT), lambda i: (0, 0)),
        ],
        out_specs=pl.BlockSpec((BLK, OUT), lambda i: (i, 0)),
        out_shape=jax.ShapeDtypeStruct((N_PAD, OUT), jnp.float32),
    )(h0h1, h0h1, Wc, bc.reshape(1, OUT))


def _sc_prop_body(cpt, h_hbm, ed_hbm, cv_hbm, out_hbm, dis_hbm,
                  X, Y, DINV2, ebuf, ebuf1, dbuf0, dbuf1, rows, rows1,
                  segb, zbuf, obuf, djbuf, djall, cv_v, gsem0, gsem1,
                  ssem0, ssem1, esem0, esem1, xsem, zsem):
    c = lax.axis_index("c")
    s = lax.axis_index("s")
    rbase = s * RPT
    obase = c * N_PAD + s * RPT
    cbase = s * cpt
    nseg = RPT // ZR

    zero16 = jnp.zeros((16,), jnp.float32)
    one16 = jnp.ones((16,), jnp.float32)

    def _zb(i, carry):
        for f in range(HALF // 16):
            zbuf[i, pl.ds(16 * f, 16)] = zero16
        return carry
    lax.fori_loop(0, ZR, _zb, 0)

    def _zo(i, carry):
        obuf[i] = zero16
        return carry
    lax.fori_loop(0, EK, _zo, 0)

    pltpu.sync_copy(cv_hbm, cv_v)
    T = cv_v[...][0]

    def _z0(j, carry):
        pltpu.sync_copy(obuf, DINV2.at[pl.ds(rbase + EK * j, EK)])
        return carry
    lax.fori_loop(0, RPT // EK, _z0, 0)

    def _ob(i, carry):
        obuf[i] = one16
        return carry
    lax.fori_loop(0, EK, _ob, 0)
    plsc.subcore_barrier()

    def _dg(ch, carry):
        pltpu.sync_copy(ed_hbm.at[cbase + ch], ebuf)
        pltpu.sync_copy(obuf, DINV2.at[ebuf.at[1]], add=True)
        return carry
    lax.fori_loop(0, cpt, _dg, 0)
    plsc.subcore_barrier()

    def _dv(j, carry):
        pltpu.sync_copy(DINV2.at[pl.ds(rbase + ZR * j, ZR)], djbuf)

        def _d1(i, cc):
            djbuf[i] = 1.0 / jnp.maximum(djbuf[i], 1.0)
            return cc
        lax.fori_loop(0, ZR, _d1, 0)
        pltpu.sync_copy(djbuf, DINV2.at[pl.ds(rbase + ZR * j, ZR)])
        return carry
    lax.fori_loop(0, nseg, _dv, 0)

    pltpu.sync_copy(DINV2.at[pl.ds(rbase, RPT)], djall)

    @pl.when(c == 0)
    def _():
        def _dout(j, carry):
            def _d2(i, cc):
                dv = djall[ZR * j + i]
                djbuf[i] = dv * _rsqrt16(dv)
                return cc
            lax.fori_loop(0, ZR, _d2, 0)
            pltpu.sync_copy(djbuf, dis_hbm.at[pl.ds(rbase + ZR * j, ZR)])
            return carry
        lax.fori_loop(0, nseg, _dout, 0)

    def _init_seg(j, carry):
        pltpu.sync_copy(h_hbm.at[pl.ds(obase + ZR * j, ZR)], segb)

        def _s0(i, cc):
            dv = djall[ZR * j + i]
            sp = dv * _rsqrt16(dv)
            for f in range(HALF // 16):
                segb[i, pl.ds(16 * f, 16)] = segb[i, pl.ds(16 * f, 16)] * sp
            return cc
        lax.fori_loop(0, ZR, _s0, 0)
        pltpu.sync_copy(segb, X.at[pl.ds(rbase + ZR * j, ZR)])
        pltpu.sync_copy(zbuf, Y.at[pl.ds(rbase + ZR * j, ZR)])
        return carry
    lax.fori_loop(0, nseg, _init_seg, 0)
    plsc.subcore_barrier()

    ebufs = [ebuf, ebuf1]
    dbufs = [dbuf0, dbuf1]
    rowss = [rows, rows1]
    gsems = [gsem0, gsem1]
    ssems = [ssem0, ssem1]
    esems = [esem0, esem1]
    npair = cpt // 2

    def _round(r, carry):
        for b in range(2):
            pltpu.async_copy(ed_hbm.at[cbase + b], ebufs[b], esems[b])

        def _pair(g, cc):
            hs = []
            for b in range(2):
                pltpu.make_async_copy(
                    ed_hbm.at[0], ebufs[b], esems[b]).wait()

                @pl.when(g > 0)
                def _(b=b):
                    pltpu.make_async_copy(
                        h_hbm.at[pl.ds(0, EK)], rowss[b], ssems[b]).wait()
                hs.append(pltpu.async_copy(
                    X.at[ebufs[b].at[0]], rowss[b], gsems[b]))
            for b in range(2):
                hs[b].wait()
                for q in range(EK // 16):
                    dbufs[b][pl.ds(16 * q, 16)] = ebufs[b][1,
                                                           pl.ds(16 * q, 16)]
                pltpu.async_copy(rowss[b], Y.at[dbufs[b]], ssems[b],
                                 add=True)

                @pl.when(g < npair - 1)
                def _(b=b, g=g):
                    pltpu.async_copy(ed_hbm.at[cbase + 2 * (g + 1) + b],
                                     ebufs[b], esems[b])
            return cc
        lax.fori_loop(0, npair, _pair, 0)
        for b in range(2):
            pltpu.make_async_copy(
                h_hbm.at[pl.ds(0, EK)], rowss[b], ssems[b]).wait()
        plsc.subcore_barrier()

        def _seg(j, cc):
            @pl.when(j > 0)
            def _():
                pltpu.make_async_copy(
                    h_hbm.at[pl.ds(0, ZR)], segb, xsem).wait()
            pltpu.sync_copy(Y.at[pl.ds(rbase + ZR * j, ZR)], segb)

            def _sr(i, c2):
                sp = djall[ZR * j + i]
                for f in range(HALF // 16):
                    segb[i, pl.ds(16 * f, 16)] = (
                        segb[i, pl.ds(16 * f, 16)] * sp)
                return c2
            lax.fori_loop(0, ZR, _sr, 0)
            pltpu.async_copy(segb, X.at[pl.ds(rbase + ZR * j, ZR)], xsem)
            pltpu.async_copy(zbuf, Y.at[pl.ds(rbase + ZR * j, ZR)], zsem)
            return cc
        lax.fori_loop(0, nseg, _seg, 0)
        pltpu.make_async_copy(h_hbm.at[pl.ds(0, ZR)], segb, xsem).wait()

        def _zdrain(j, cc):
            pltpu.make_async_copy(h_hbm.at[pl.ds(0, ZR)], zbuf, zsem).wait()
            return cc
        lax.fori_loop(0, nseg, _zdrain, 0)
        plsc.subcore_barrier()
        return carry
    lax.fori_loop(0, T, _round, 0)

    def _out_seg(j, carry):
        pltpu.sync_copy(X.at[pl.ds(rbase + ZR * j, ZR)], segb)

        def _so(i, cc):
            sp = _rsqrt16(djall[ZR * j + i])
            for f in range(HALF // 16):
                segb[i, pl.ds(16 * f, 16)] = segb[i, pl.ds(16 * f, 16)] * sp
            return cc
        lax.fori_loop(0, ZR, _so, 0)
        pltpu.sync_copy(segb, out_hbm.at[pl.ds(obase + ZR * j, ZR)])
        return carry
    lax.fori_loop(0, nseg, _out_seg, 0)


def _sc_prop(h2, edges, cv, nch):
    cpt = nch // 16
    f = pl.kernel(
        functools.partial(_sc_prop_body, cpt),
        out_type=[jax.ShapeDtypeStruct((2 * N_PAD, HALF), jnp.float32),
                  jax.ShapeDtypeStruct((N_PAD, 16), jnp.float32)],
        mesh=plsc.VectorSubcoreMesh(**_MESH),
        compiler_params=pltpu.CompilerParams(use_tc_tiling_on_sc=False),
        scratch_types=[
            pltpu.VMEM_SHARED((N_PAD, HALF), jnp.float32),
            pltpu.VMEM_SHARED((N_PAD, HALF), jnp.float32),
            pltpu.VMEM_SHARED((N_PAD, 16), jnp.float32),
            pltpu.VMEM((2, EK), jnp.int32),
            pltpu.VMEM((2, EK), jnp.int32),
            pltpu.VMEM((EK,), jnp.int32),
            pltpu.VMEM((EK,), jnp.int32),
            pltpu.VMEM((EK, HALF), jnp.float32),
            pltpu.VMEM((EK, HALF), jnp.float32),
            pltpu.VMEM((ZR, HALF), jnp.float32),
            pltpu.VMEM((ZR, HALF), jnp.float32),
            pltpu.VMEM((EK, 16), jnp.float32),
            pltpu.VMEM((ZR, 16), jnp.float32),
            pltpu.VMEM((RPT, 16), jnp.float32),
            pltpu.VMEM((16,), jnp.int32),
            pltpu.SemaphoreType.DMA,
            pltpu.SemaphoreType.DMA,
            pltpu.SemaphoreType.DMA,
            pltpu.SemaphoreType.DMA,
            pltpu.SemaphoreType.DMA,
            pltpu.SemaphoreType.DMA,
            pltpu.SemaphoreType.DMA,
            pltpu.SemaphoreType.DMA,
        ],
    )
    return f(h2, edges, cv)


def _sc_final_body(cpt, h_hbm, ed_hbm, dis_hbm, y_hbm,
                   X16, Y16, ebuf, ebuf1, dbuf0, dbuf1, rows16, rows16b,
                   rowbuf16, zb16, dis2b,
                   gsem0, gsem1, ssem0, ssem1, esem0, esem1):
    c = lax.axis_index("c")
    s = lax.axis_index("s")
    rbase = s * RPT
    cbase = s * cpt

    @pl.when(c == 0)
    def _():
        zero16 = jnp.zeros((16,), jnp.float32)

        def _zb(i, carry):
            zb16[i] = zero16
            return carry
        lax.fori_loop(0, ZR, _zb, 0)

        pltpu.sync_copy(dis_hbm.at[pl.ds(rbase, RPT)], dis2b)
        pltpu.sync_copy(h_hbm.at[pl.ds(rbase, RPT)], rowbuf16)

        def _s0(i, carry):
            rowbuf16[i] = rowbuf16[i] * dis2b[i]
            return carry
        lax.fori_loop(0, RPT, _s0, 0)
        pltpu.sync_copy(rowbuf16, X16.at[pl.ds(rbase, RPT)])

        def _zy(j, carry):
            pltpu.sync_copy(zb16, Y16.at[pl.ds(rbase + ZR * j, ZR)])
            return carry
        lax.fori_loop(0, RPT // ZR, _zy, 0)
        plsc.subcore_barrier()

        ebufs = [ebuf, ebuf1]
        dbufs = [dbuf0, dbuf1]
        rowss = [rows16, rows16b]
        gsems = [gsem0, gsem1]
        ssems = [ssem0, ssem1]
        esems = [esem0, esem1]
        npair = cpt // 2
        for b in range(2):
            pltpu.async_copy(ed_hbm.at[cbase + b], ebufs[b], esems[b])

        def _pair(g, carry):
            hs = []
            for b in range(2):
                pltpu.make_async_copy(
                    ed_hbm.at[0], ebufs[b], esems[b]).wait()

                @pl.when(g > 0)
                def _(b=b):
                    pltpu.make_async_copy(
                        dis_hbm.at[pl.ds(0, EK)], rowss[b], ssems[b]).wait()
                hs.append(pltpu.async_copy(
                    X16.at[ebufs[b].at[0]], rowss[b], gsems[b]))
            for b in range(2):
                hs[b].wait()
                for q in range(EK // 16):
                    dbufs[b][pl.ds(16 * q, 16)] = ebufs[b][1,
                                                           pl.ds(16 * q, 16)]
                pltpu.async_copy(rowss[b], Y16.at[dbufs[b]], ssems[b],
                                 add=True)

                @pl.when(g < npair - 1)
                def _(b=b, g=g):
                    pltpu.async_copy(ed_hbm.at[cbase + 2 * (g + 1) + b],
                                     ebufs[b], esems[b])
            return carry
        lax.fori_loop(0, npair, _pair, 0)
        for b in range(2):
            pltpu.make_async_copy(
                dis_hbm.at[pl.ds(0, EK)], rowss[b], ssems[b]).wait()
        plsc.subcore_barrier()

        pltpu.sync_copy(Y16.at[pl.ds(rbase, RPT)], rowbuf16)

        def _s1(i, carry):
            rowbuf16[i] = rowbuf16[i] * dis2b[i]
            return carry
        lax.fori_loop(0, RPT, _s1, 0)
        pltpu.sync_copy(rowbuf16, y_hbm.at[pl.ds(rbase, RPT)])


def _sc_final(h3, edges, dis, nch):
    cpt = nch // 16
    f = pl.kernel(
        functools.partial(_sc_final_body, cpt),
        out_type=jax.ShapeDtypeStruct((N_PAD, OUT), jnp.float32),
        mesh=plsc.VectorSubcoreMesh(**_MESH),
        compiler_params=pltpu.CompilerParams(use_tc_tiling_on_sc=False),
        scratch_types=[
            pltpu.VMEM_SHARED((N_PAD, OUT), jnp.float32),
            pltpu.VMEM_SHARED((N_PAD, OUT), jnp.float32),
            pltpu.VMEM((2, EK), jnp.int32),
            pltpu.VMEM((2, EK), jnp.int32),
            pltpu.VMEM((EK,), jnp.int32),
            pltpu.VMEM((EK,), jnp.int32),
            pltpu.VMEM((EK, OUT), jnp.float32),
            pltpu.VMEM((EK, OUT), jnp.float32),
            pltpu.VMEM((RPT, OUT), jnp.float32),
            pltpu.VMEM((ZR, OUT), jnp.float32),
            pltpu.VMEM((RPT, 16), jnp.float32),
            pltpu.SemaphoreType.DMA,
            pltpu.SemaphoreType.DMA,
            pltpu.SemaphoreType.DMA,
            pltpu.SemaphoreType.DMA,
            pltpu.SemaphoreType.DMA,
            pltpu.SemaphoreType.DMA,
        ],
    )
    return f(h3, edges, dis)


def kernel(x, edge_index, W1, b1, W2, b2, Wc, bc, conv_time):
    N = x.shape[0]
    E = edge_index.shape[1]
    ET = E + N
    nch = -(-ET // EK)
    nch = -(-nch // 16) * 16
    epad = nch * EK

    loop = jnp.arange(N, dtype=jnp.int32)
    padv = jnp.full((epad - ET,), N, jnp.int32)
    src = jnp.concatenate([edge_index[0].astype(jnp.int32), loop, padv])
    dst = jnp.concatenate([edge_index[1].astype(jnp.int32), loop, padv])
    edges = jnp.stack([src.reshape(nch, EK), dst.reshape(nch, EK)], axis=1)

    x_pad = jnp.pad(x, ((0, N_PAD - N), (0, 0)))
    cv = jnp.full((16,), conv_time, jnp.int32)

    h2 = _tc_front(x_pad, W1, b1, W2, b2)
    hp, dis = _sc_prop(h2, edges, cv, nch)
    h3 = _tc_cls(hp, Wc, bc)
    y = _sc_final(h3, edges, dis, nch)
    return y[:N]


# speedup vs baseline: 1.4026x; 1.0135x over previous
import functools

import jax
import jax.numpy as jnp
from jax import lax
from jax.experimental import pallas as pl
from jax.experimental.pallas import tpu as pltpu
from jax.experimental.pallas import tpu_sc as plsc

N_PAD = 10240
RPT = N_PAD // 16
ZR = 64
EK = 128
HID = 128
HALF = 64
OUT = 16

_MESH = dict(core_axis_name="c", subcore_axis_name="s", num_cores=2,
             num_subcores=16)


def _rsqrt16(v):
    bits = lax.bitcast_convert_type(v, jnp.int32)
    y = lax.bitcast_convert_type(0x5F3759DF - (bits >> 1), jnp.float32)
    for _ in range(4):
        y = y * (1.5 - 0.5 * v * y * y)
    return y


def _tc_front(x_pad, W1, b1, W2, b2):
    BLK = 256

    def body(x_ref, w1_ref, b1_ref, w2_ref, b2_ref, o_ref):
        h = jnp.dot(x_ref[...], w1_ref[...],
                    preferred_element_type=jnp.float32) + b1_ref[...]
        h = jnp.maximum(h, 0.0)
        h = jnp.dot(h, w2_ref[...],
                    preferred_element_type=jnp.float32) + b2_ref[...]
        o_ref[0] = h[:, :HALF]
        o_ref[1] = h[:, HALF:]

    out = pl.pallas_call(
        body,
        grid=(N_PAD // BLK,),
        in_specs=[
            pl.BlockSpec((BLK, HID), lambda i: (i, 0)),
            pl.BlockSpec((HID, HID), lambda i: (0, 0)),
            pl.BlockSpec((1, HID), lambda i: (0, 0)),
            pl.BlockSpec((HID, HID), lambda i: (0, 0)),
            pl.BlockSpec((1, HID), lambda i: (0, 0)),
        ],
        out_specs=pl.BlockSpec((2, BLK, HALF), lambda i: (0, i, 0)),
        out_shape=jax.ShapeDtypeStruct((2, N_PAD, HALF), jnp.float32),
    )(x_pad, W1, b1.reshape(1, HID), W2, b2.reshape(1, HID))
    return out.reshape(2 * N_PAD, HALF)


def _tc_cls(h0h1, Wc, bc):
    BLK = 256
    nblk = N_PAD // BLK

    def body(a_ref, b_ref, wc_ref, bc_ref, o_ref):
        h = jnp.concatenate([a_ref[...], b_ref[...]], axis=1)
        h = jnp.maximum(h, 0.0)
        o_ref[...] = jnp.dot(h, wc_ref[...],
                             preferred_element_type=jnp.float32) + bc_ref[...]

    return pl.pallas_call(
        body,
        grid=(nblk,),
        in_specs=[
            pl.BlockSpec((BLK, HALF), lambda i: (i, 0)),
            pl.BlockSpec((BLK, HALF), lambda i: (i + nblk, 0)),
            pl.BlockSpec((HID, OUT), lambda i: (0, 0)),
            pl.BlockSpec((1, OUT), lambda i: (0, 0)),
        ],
        out_specs=pl.BlockSpec((BLK, OUT), lambda i: (i, 0)),
        out_shape=jax.ShapeDtypeStruct((N_PAD, OUT), jnp.float32),
    )(h0h1, h0h1, Wc, bc.reshape(1, OUT))


def _sc_prop_body(cpt, h_hbm, ed_hbm, cv_hbm, out_hbm, dis_hbm,
                  X, Y, DINV2, ebuf, ebuf1, dbuf0, dbuf1, rows, rows1,
                  segb, zbuf, obuf, djbuf, djall, cv_v, gsem0, gsem1,
                  ssem0, ssem1, esem0, esem1, xsem, zsem):
    c = lax.axis_index("c")
    s = lax.axis_index("s")
    rbase = s * RPT
    obase = c * N_PAD + s * RPT
    cbase = s * cpt
    nseg = RPT // ZR

    zero16 = jnp.zeros((16,), jnp.float32)
    one16 = jnp.ones((16,), jnp.float32)

    def _zb(i, carry):
        for f in range(HALF // 16):
            zbuf[i, pl.ds(16 * f, 16)] = zero16
        return carry
    lax.fori_loop(0, ZR, _zb, 0)

    def _zo(i, carry):
        obuf[i] = zero16
        return carry
    lax.fori_loop(0, EK, _zo, 0)

    pltpu.sync_copy(cv_hbm, cv_v)
    T = cv_v[...][0]

    def _z0(j, carry):
        pltpu.sync_copy(obuf, DINV2.at[pl.ds(rbase + EK * j, EK)])
        return carry
    lax.fori_loop(0, RPT // EK, _z0, 0)

    def _ob(i, carry):
        obuf[i] = one16
        return carry
    lax.fori_loop(0, EK, _ob, 0)
    plsc.subcore_barrier()

    for b, eb, es in ((0, ebuf, esem0), (1, ebuf1, esem1)):
        pltpu.async_copy(ed_hbm.at[cbase + b], eb, esem0 if b == 0 else esem1)

    def _dg(g, carry):
        for b, eb, db, es, ss in ((0, ebuf, dbuf0, esem0, ssem0),
                                  (1, ebuf1, dbuf1, esem1, ssem1)):
            pltpu.make_async_copy(ed_hbm.at[0], eb, es).wait()

            @pl.when(g > 0)
            def _(ss=ss):
                pltpu.make_async_copy(
                    dis_hbm.at[pl.ds(0, EK)], obuf, ss).wait()
            for q in range(EK // 16):
                db[pl.ds(16 * q, 16)] = eb[1, pl.ds(16 * q, 16)]
            pltpu.async_copy(obuf, DINV2.at[db], ss, add=True)

            @pl.when(g < cpt // 2 - 1)
            def _(b=b, eb=eb, es=es, g=g):
                pltpu.async_copy(ed_hbm.at[cbase + 2 * (g + 1) + b], eb, es)
        return carry
    lax.fori_loop(0, cpt // 2, _dg, 0)
    for ss in (ssem0, ssem1):
        pltpu.make_async_copy(dis_hbm.at[pl.ds(0, EK)], obuf, ss).wait()
    plsc.subcore_barrier()

    def _dv(j, carry):
        pltpu.sync_copy(DINV2.at[pl.ds(rbase + ZR * j, ZR)], djbuf)

        def _d1(i, cc):
            djbuf[i] = 1.0 / jnp.maximum(djbuf[i], 1.0)
            return cc
        lax.fori_loop(0, ZR, _d1, 0)
        pltpu.sync_copy(djbuf, DINV2.at[pl.ds(rbase + ZR * j, ZR)])
        return carry
    lax.fori_loop(0, nseg, _dv, 0)

    pltpu.sync_copy(DINV2.at[pl.ds(rbase, RPT)], djall)

    @pl.when(c == 0)
    def _():
        def _dout(j, carry):
            def _d2(i, cc):
                dv = djall[ZR * j + i]
                djbuf[i] = dv * _rsqrt16(dv)
                return cc
            lax.fori_loop(0, ZR, _d2, 0)
            pltpu.sync_copy(djbuf, dis_hbm.at[pl.ds(rbase + ZR * j, ZR)])
            return carry
        lax.fori_loop(0, nseg, _dout, 0)

    def _init_seg(j, carry):
        pltpu.sync_copy(h_hbm.at[pl.ds(obase + ZR * j, ZR)], segb)

        def _s0(i, cc):
            dv = djall[ZR * j + i]
            sp = dv * _rsqrt16(dv)
            for f in range(HALF // 16):
                segb[i, pl.ds(16 * f, 16)] = segb[i, pl.ds(16 * f, 16)] * sp
            return cc
        lax.fori_loop(0, ZR, _s0, 0)
        pltpu.sync_copy(segb, X.at[pl.ds(rbase + ZR * j, ZR)])
        pltpu.sync_copy(zbuf, Y.at[pl.ds(rbase + ZR * j, ZR)])
        return carry
    lax.fori_loop(0, nseg, _init_seg, 0)
    plsc.subcore_barrier()

    ebufs = [ebuf, ebuf1]
    dbufs = [dbuf0, dbuf1]
    rowss = [rows, rows1]
    gsems = [gsem0, gsem1]
    ssems = [ssem0, ssem1]
    esems = [esem0, esem1]
    npair = cpt // 2

    def _round(r, carry):
        for b in range(2):
            pltpu.async_copy(ed_hbm.at[cbase + b], ebufs[b], esems[b])

        def _pair(g, cc):
            hs = []
            for b in range(2):
                pltpu.make_async_copy(
                    ed_hbm.at[0], ebufs[b], esems[b]).wait()

                @pl.when(g > 0)
                def _(b=b):
                    pltpu.make_async_copy(
                        h_hbm.at[pl.ds(0, EK)], rowss[b], ssems[b]).wait()
                hs.append(pltpu.async_copy(
                    X.at[ebufs[b].at[0]], rowss[b], gsems[b]))
            for b in range(2):
                hs[b].wait()
                for q in range(EK // 16):
                    dbufs[b][pl.ds(16 * q, 16)] = ebufs[b][1,
                                                           pl.ds(16 * q, 16)]
                pltpu.async_copy(rowss[b], Y.at[dbufs[b]], ssems[b],
                                 add=True)

                @pl.when(g < npair - 1)
                def _(b=b, g=g):
                    pltpu.async_copy(ed_hbm.at[cbase + 2 * (g + 1) + b],
                                     ebufs[b], esems[b])
            return cc
        lax.fori_loop(0, npair, _pair, 0)
        for b in range(2):
            pltpu.make_async_copy(
                h_hbm.at[pl.ds(0, EK)], rowss[b], ssems[b]).wait()
        plsc.subcore_barrier()

        def _seg(j, cc):
            @pl.when(j > 0)
            def _():
                pltpu.make_async_copy(
                    h_hbm.at[pl.ds(0, ZR)], segb, xsem).wait()
            pltpu.sync_copy(Y.at[pl.ds(rbase + ZR * j, ZR)], segb)

            def _sr(i, c2):
                sp = djall[ZR * j + i]
                for f in range(HALF // 16):
                    segb[i, pl.ds(16 * f, 16)] = (
                        segb[i, pl.ds(16 * f, 16)] * sp)
                return c2
            lax.fori_loop(0, ZR, _sr, 0)
            pltpu.async_copy(segb, X.at[pl.ds(rbase + ZR * j, ZR)], xsem)
            pltpu.async_copy(zbuf, Y.at[pl.ds(rbase + ZR * j, ZR)], zsem)
            return cc
        lax.fori_loop(0, nseg, _seg, 0)
        pltpu.make_async_copy(h_hbm.at[pl.ds(0, ZR)], segb, xsem).wait()

        def _zdrain(j, cc):
            pltpu.make_async_copy(h_hbm.at[pl.ds(0, ZR)], zbuf, zsem).wait()
            return cc
        lax.fori_loop(0, nseg, _zdrain, 0)
        plsc.subcore_barrier()
        return carry
    lax.fori_loop(0, T, _round, 0)

    def _out_seg(j, carry):
        pltpu.sync_copy(X.at[pl.ds(rbase + ZR * j, ZR)], segb)

        def _so(i, cc):
            sp = _rsqrt16(djall[ZR * j + i])
            for f in range(HALF // 16):
                segb[i, pl.ds(16 * f, 16)] = segb[i, pl.ds(16 * f, 16)] * sp
            return cc
        lax.fori_loop(0, ZR, _so, 0)
        pltpu.sync_copy(segb, out_hbm.at[pl.ds(obase + ZR * j, ZR)])
        return carry
    lax.fori_loop(0, nseg, _out_seg, 0)


def _sc_prop(h2, edges, cv, nch):
    cpt = nch // 16
    f = pl.kernel(
        functools.partial(_sc_prop_body, cpt),
        out_type=[jax.ShapeDtypeStruct((2 * N_PAD, HALF), jnp.float32),
                  jax.ShapeDtypeStruct((N_PAD, 16), jnp.float32)],
        mesh=plsc.VectorSubcoreMesh(**_MESH),
        compiler_params=pltpu.CompilerParams(use_tc_tiling_on_sc=False),
        scratch_types=[
            pltpu.VMEM_SHARED((N_PAD, HALF), jnp.float32),
            pltpu.VMEM_SHARED((N_PAD, HALF), jnp.float32),
            pltpu.VMEM_SHARED((N_PAD, 16), jnp.float32),
            pltpu.VMEM((2, EK), jnp.int32),
            pltpu.VMEM((2, EK), jnp.int32),
            pltpu.VMEM((EK,), jnp.int32),
            pltpu.VMEM((EK,), jnp.int32),
            pltpu.VMEM((EK, HALF), jnp.float32),
            pltpu.VMEM((EK, HALF), jnp.float32),
            pltpu.VMEM((ZR, HALF), jnp.float32),
            pltpu.VMEM((ZR, HALF), jnp.float32),
            pltpu.VMEM((EK, 16), jnp.float32),
            pltpu.VMEM((ZR, 16), jnp.float32),
            pltpu.VMEM((RPT, 16), jnp.float32),
            pltpu.VMEM((16,), jnp.int32),
            pltpu.SemaphoreType.DMA,
            pltpu.SemaphoreType.DMA,
            pltpu.SemaphoreType.DMA,
            pltpu.SemaphoreType.DMA,
            pltpu.SemaphoreType.DMA,
            pltpu.SemaphoreType.DMA,
            pltpu.SemaphoreType.DMA,
            pltpu.SemaphoreType.DMA,
        ],
    )
    return f(h2, edges, cv)


def _sc_final_body(cpt, h_hbm, ed_hbm, dis_hbm, y_hbm,
                   X16, Y16, ebuf, ebuf1, dbuf0, dbuf1, rows16, rows16b,
                   rowbuf16, zb16, dis2b,
                   gsem0, gsem1, ssem0, ssem1, esem0, esem1):
    c = lax.axis_index("c")
    s = lax.axis_index("s")
    rbase = s * RPT
    cbase = s * cpt

    @pl.when(c == 0)
    def _():
        zero16 = jnp.zeros((16,), jnp.float32)

        def _zb(i, carry):
            zb16[i] = zero16
            return carry
        lax.fori_loop(0, ZR, _zb, 0)

        pltpu.sync_copy(dis_hbm.at[pl.ds(rbase, RPT)], dis2b)
        pltpu.sync_copy(h_hbm.at[pl.ds(rbase, RPT)], rowbuf16)

        def _s0(i, carry):
            rowbuf16[i] = rowbuf16[i] * dis2b[i]
            return carry
        lax.fori_loop(0, RPT, _s0, 0)
        pltpu.sync_copy(rowbuf16, X16.at[pl.ds(rbase, RPT)])

        def _zy(j, carry):
            pltpu.sync_copy(zb16, Y16.at[pl.ds(rbase + ZR * j, ZR)])
            return carry
        lax.fori_loop(0, RPT // ZR, _zy, 0)
        plsc.subcore_barrier()

        ebufs = [ebuf, ebuf1]
        dbufs = [dbuf0, dbuf1]
        rowss = [rows16, rows16b]
        gsems = [gsem0, gsem1]
        ssems = [ssem0, ssem1]
        esems = [esem0, esem1]
        npair = cpt // 2
        for b in range(2):
            pltpu.async_copy(ed_hbm.at[cbase + b], ebufs[b], esems[b])

        def _pair(g, carry):
            hs = []
            for b in range(2):
                pltpu.make_async_copy(
                    ed_hbm.at[0], ebufs[b], esems[b]).wait()

                @pl.when(g > 0)
                def _(b=b):
                    pltpu.make_async_copy(
                        dis_hbm.at[pl.ds(0, EK)], rowss[b], ssems[b]).wait()
                hs.append(pltpu.async_copy(
                    X16.at[ebufs[b].at[0]], rowss[b], gsems[b]))
            for b in range(2):
                hs[b].wait()
                for q in range(EK // 16):
                    dbufs[b][pl.ds(16 * q, 16)] = ebufs[b][1,
                                                           pl.ds(16 * q, 16)]
                pltpu.async_copy(rowss[b], Y16.at[dbufs[b]], ssems[b],
                                 add=True)

                @pl.when(g < npair - 1)
                def _(b=b, g=g):
                    pltpu.async_copy(ed_hbm.at[cbase + 2 * (g + 1) + b],
                                     ebufs[b], esems[b])
            return carry
        lax.fori_loop(0, npair, _pair, 0)
        for b in range(2):
            pltpu.make_async_copy(
                dis_hbm.at[pl.ds(0, EK)], rowss[b], ssems[b]).wait()
        plsc.subcore_barrier()

        pltpu.sync_copy(Y16.at[pl.ds(rbase, RPT)], rowbuf16)

        def _s1(i, carry):
            rowbuf16[i] = rowbuf16[i] * dis2b[i]
            return carry
        lax.fori_loop(0, RPT, _s1, 0)
        pltpu.sync_copy(rowbuf16, y_hbm.at[pl.ds(rbase, RPT)])


def _sc_final(h3, edges, dis, nch):
    cpt = nch // 16
    f = pl.kernel(
        functools.partial(_sc_final_body, cpt),
        out_type=jax.ShapeDtypeStruct((N_PAD, OUT), jnp.float32),
        mesh=plsc.VectorSubcoreMesh(**_MESH),
        compiler_params=pltpu.CompilerParams(use_tc_tiling_on_sc=False),
        scratch_types=[
            pltpu.VMEM_SHARED((N_PAD, OUT), jnp.float32),
            pltpu.VMEM_SHARED((N_PAD, OUT), jnp.float32),
            pltpu.VMEM((2, EK), jnp.int32),
            pltpu.VMEM((2, EK), jnp.int32),
            pltpu.VMEM((EK,), jnp.int32),
            pltpu.VMEM((EK,), jnp.int32),
            pltpu.VMEM((EK, OUT), jnp.float32),
            pltpu.VMEM((EK, OUT), jnp.float32),
            pltpu.VMEM((RPT, OUT), jnp.float32),
            pltpu.VMEM((ZR, OUT), jnp.float32),
            pltpu.VMEM((RPT, 16), jnp.float32),
            pltpu.SemaphoreType.DMA,
            pltpu.SemaphoreType.DMA,
            pltpu.SemaphoreType.DMA,
            pltpu.SemaphoreType.DMA,
            pltpu.SemaphoreType.DMA,
            pltpu.SemaphoreType.DMA,
        ],
    )
    return f(h3, edges, dis)


def kernel(x, edge_index, W1, b1, W2, b2, Wc, bc, conv_time):
    N = x.shape[0]
    E = edge_index.shape[1]
    ET = E + N
    nch = -(-ET // EK)
    nch = -(-nch // 16) * 16
    epad = nch * EK

    loop = jnp.arange(N, dtype=jnp.int32)
    padv = jnp.full((epad - ET,), N, jnp.int32)
    src = jnp.concatenate([edge_index[0].astype(jnp.int32), loop, padv])
    dst = jnp.concatenate([edge_index[1].astype(jnp.int32), loop, padv])
    edges = jnp.stack([src.reshape(nch, EK), dst.reshape(nch, EK)], axis=1)

    x_pad = jnp.pad(x, ((0, N_PAD - N), (0, 0)))
    cv = jnp.full((16,), conv_time, jnp.int32)

    h2 = _tc_front(x_pad, W1, b1, W2, b2)
    hp, dis = _sc_prop(h2, edges, cv, nch)
    h3 = _tc_cls(hp, Wc, bc)
    y = _sc_final(h3, edges, dis, nch)
    return y[:N]


# speedup vs baseline: 1.4242x; 1.0154x over previous
import functools

import jax
import jax.numpy as jnp
from jax import lax
from jax.experimental import pallas as pl
from jax.experimental.pallas import tpu as pltpu
from jax.experimental.pallas import tpu_sc as plsc

N_PAD = 10240
RPT = N_PAD // 16
ZR = 64
EK = 128
HID = 128
HALF = 64
OUT = 16

_MESH = dict(core_axis_name="c", subcore_axis_name="s", num_cores=2,
             num_subcores=16)


def _rsqrt16(v):
    bits = lax.bitcast_convert_type(v, jnp.int32)
    y = lax.bitcast_convert_type(0x5F3759DF - (bits >> 1), jnp.float32)
    for _ in range(4):
        y = y * (1.5 - 0.5 * v * y * y)
    return y


def _tc_front(x_pad, W1, b1, W2, b2):
    BLK = 256

    def body(x_ref, w1_ref, b1_ref, w2_ref, b2_ref, o_ref):
        h = jnp.dot(x_ref[...], w1_ref[...],
                    preferred_element_type=jnp.float32) + b1_ref[...]
        h = jnp.maximum(h, 0.0)
        h = jnp.dot(h, w2_ref[...],
                    preferred_element_type=jnp.float32) + b2_ref[...]
        o_ref[0] = h[:, :HALF]
        o_ref[1] = h[:, HALF:]

    out = pl.pallas_call(
        body,
        grid=(N_PAD // BLK,),
        in_specs=[
            pl.BlockSpec((BLK, HID), lambda i: (i, 0)),
            pl.BlockSpec((HID, HID), lambda i: (0, 0)),
            pl.BlockSpec((1, HID), lambda i: (0, 0)),
            pl.BlockSpec((HID, HID), lambda i: (0, 0)),
            pl.BlockSpec((1, HID), lambda i: (0, 0)),
        ],
        out_specs=pl.BlockSpec((2, BLK, HALF), lambda i: (0, i, 0)),
        out_shape=jax.ShapeDtypeStruct((2, N_PAD, HALF), jnp.float32),
    )(x_pad, W1, b1.reshape(1, HID), W2, b2.reshape(1, HID))
    return out.reshape(2 * N_PAD, HALF)


def _tc_cls(h0h1, Wc, bc):
    BLK = 256
    nblk = N_PAD // BLK

    def body(a_ref, b_ref, wc_ref, bc_ref, o_ref):
        h = jnp.concatenate([a_ref[...], b_ref[...]], axis=1)
        h = jnp.maximum(h, 0.0)
        o_ref[...] = jnp.dot(h, wc_ref[...],
                             preferred_element_type=jnp.float32) + bc_ref[...]

    return pl.pallas_call(
        body,
        grid=(nblk,),
        in_specs=[
            pl.BlockSpec((BLK, HALF), lambda i: (i, 0)),
            pl.BlockSpec((BLK, HALF), lambda i: (i + nblk, 0)),
            pl.BlockSpec((HID, OUT), lambda i: (0, 0)),
            pl.BlockSpec((1, OUT), lambda i: (0, 0)),
        ],
        out_specs=pl.BlockSpec((BLK, OUT), lambda i: (i, 0)),
        out_shape=jax.ShapeDtypeStruct((N_PAD, OUT), jnp.float32),
    )(h0h1, h0h1, Wc, bc.reshape(1, OUT))


def _sc_prop_body(cpt, h_hbm, ed_hbm, cv_hbm, out_hbm, dis_hbm,
                  X, Y, DINV2, ebuf, ebuf1, dbuf0, dbuf1, rows, rows1,
                  segb, obuf, djbuf, djall, cv_v, gsem0, gsem1,
                  ssem0, ssem1, esem0, esem1, xsem, zsem):
    c = lax.axis_index("c")
    s = lax.axis_index("s")
    rbase = s * RPT
    obase = c * N_PAD + s * RPT
    cbase = s * cpt
    nseg = RPT // ZR

    zero16 = jnp.zeros((16,), jnp.float32)
    one16 = jnp.ones((16,), jnp.float32)

    def _zo(i, carry):
        obuf[i] = zero16
        return carry
    lax.fori_loop(0, EK, _zo, 0)

    pltpu.sync_copy(cv_hbm, cv_v)
    T = cv_v[...][0]

    def _z0(j, carry):
        pltpu.sync_copy(obuf, DINV2.at[pl.ds(rbase + EK * j, EK)])
        return carry
    lax.fori_loop(0, RPT // EK, _z0, 0)

    def _ob(i, carry):
        obuf[i] = one16
        return carry
    lax.fori_loop(0, EK, _ob, 0)
    plsc.subcore_barrier()

    for b, eb, es in ((0, ebuf, esem0), (1, ebuf1, esem1)):
        pltpu.async_copy(ed_hbm.at[cbase + b], eb, esem0 if b == 0 else esem1)

    def _dg(g, carry):
        for b, eb, db, es, ss in ((0, ebuf, dbuf0, esem0, ssem0),
                                  (1, ebuf1, dbuf1, esem1, ssem1)):
            pltpu.make_async_copy(ed_hbm.at[0], eb, es).wait()

            @pl.when(g > 0)
            def _(ss=ss):
                pltpu.make_async_copy(
                    dis_hbm.at[pl.ds(0, EK)], obuf, ss).wait()
            for q in range(EK // 16):
                db[pl.ds(16 * q, 16)] = eb[1, pl.ds(16 * q, 16)]
            pltpu.async_copy(obuf, DINV2.at[db], ss, add=True)

            @pl.when(g < cpt // 2 - 1)
            def _(b=b, eb=eb, es=es, g=g):
                pltpu.async_copy(ed_hbm.at[cbase + 2 * (g + 1) + b], eb, es)
        return carry
    lax.fori_loop(0, cpt // 2, _dg, 0)
    for ss in (ssem0, ssem1):
        pltpu.make_async_copy(dis_hbm.at[pl.ds(0, EK)], obuf, ss).wait()
    plsc.subcore_barrier()

    def _dv(j, carry):
        pltpu.sync_copy(DINV2.at[pl.ds(rbase + ZR * j, ZR)], djbuf)

        def _d1(i, cc):
            djbuf[i] = 1.0 / (djbuf[i] + 1.0)
            return cc
        lax.fori_loop(0, ZR, _d1, 0)
        pltpu.sync_copy(djbuf, DINV2.at[pl.ds(rbase + ZR * j, ZR)])
        return carry
    lax.fori_loop(0, nseg, _dv, 0)

    pltpu.sync_copy(DINV2.at[pl.ds(rbase, RPT)], djall)

    @pl.when(c == 0)
    def _():
        def _dout(j, carry):
            def _d2(i, cc):
                dv = djall[ZR * j + i]
                djbuf[i] = dv * _rsqrt16(dv)
                return cc
            lax.fori_loop(0, ZR, _d2, 0)
            pltpu.sync_copy(djbuf, dis_hbm.at[pl.ds(rbase + ZR * j, ZR)])
            return carry
        lax.fori_loop(0, nseg, _dout, 0)

    def _init_seg(j, carry):
        pltpu.sync_copy(h_hbm.at[pl.ds(obase + ZR * j, ZR)], segb)

        def _s0(i, cc):
            dv = djall[ZR * j + i]
            sp = dv * _rsqrt16(dv)
            for f in range(HALF // 16):
                segb[i, pl.ds(16 * f, 16)] = segb[i, pl.ds(16 * f, 16)] * sp
            return cc
        lax.fori_loop(0, ZR, _s0, 0)
        pltpu.sync_copy(segb, X.at[pl.ds(rbase + ZR * j, ZR)])
        pltpu.sync_copy(segb, Y.at[pl.ds(rbase + ZR * j, ZR)])
        return carry
    lax.fori_loop(0, nseg, _init_seg, 0)
    plsc.subcore_barrier()

    ebufs = [ebuf, ebuf1]
    dbufs = [dbuf0, dbuf1]
    rowss = [rows, rows1]
    gsems = [gsem0, gsem1]
    ssems = [ssem0, ssem1]
    esems = [esem0, esem1]
    npair = cpt // 2

    def _round(r, carry):
        for b in range(2):
            pltpu.async_copy(ed_hbm.at[cbase + b], ebufs[b], esems[b])

        def _pair(g, cc):
            hs = []
            for b in range(2):
                pltpu.make_async_copy(
                    ed_hbm.at[0], ebufs[b], esems[b]).wait()

                @pl.when(g > 0)
                def _(b=b):
                    pltpu.make_async_copy(
                        h_hbm.at[pl.ds(0, EK)], rowss[b], ssems[b]).wait()
                hs.append(pltpu.async_copy(
                    X.at[ebufs[b].at[0]], rowss[b], gsems[b]))
            for b in range(2):
                hs[b].wait()
                for q in range(EK // 16):
                    dbufs[b][pl.ds(16 * q, 16)] = ebufs[b][1,
                                                           pl.ds(16 * q, 16)]
                pltpu.async_copy(rowss[b], Y.at[dbufs[b]], ssems[b],
                                 add=True)

                @pl.when(g < npair - 1)
                def _(b=b, g=g):
                    pltpu.async_copy(ed_hbm.at[cbase + 2 * (g + 1) + b],
                                     ebufs[b], esems[b])
            return cc
        lax.fori_loop(0, npair, _pair, 0)
        for b in range(2):
            pltpu.make_async_copy(
                h_hbm.at[pl.ds(0, EK)], rowss[b], ssems[b]).wait()
        plsc.subcore_barrier()

        def _seg(j, cc):
            @pl.when(j > 0)
            def _():
                pltpu.make_async_copy(
                    h_hbm.at[pl.ds(0, ZR)], segb, xsem).wait()
                pltpu.make_async_copy(
                    h_hbm.at[pl.ds(0, ZR)], segb, zsem).wait()
            pltpu.sync_copy(Y.at[pl.ds(rbase + ZR * j, ZR)], segb)

            def _sr(i, c2):
                sp = djall[ZR * j + i]
                for f in range(HALF // 16):
                    segb[i, pl.ds(16 * f, 16)] = (
                        segb[i, pl.ds(16 * f, 16)] * sp)
                return c2
            lax.fori_loop(0, ZR, _sr, 0)
            pltpu.async_copy(segb, X.at[pl.ds(rbase + ZR * j, ZR)], xsem)
            pltpu.async_copy(segb, Y.at[pl.ds(rbase + ZR * j, ZR)], zsem)
            return cc
        lax.fori_loop(0, nseg, _seg, 0)
        pltpu.make_async_copy(h_hbm.at[pl.ds(0, ZR)], segb, xsem).wait()
        pltpu.make_async_copy(h_hbm.at[pl.ds(0, ZR)], segb, zsem).wait()
        plsc.subcore_barrier()
        return carry
    lax.fori_loop(0, T, _round, 0)

    def _out_seg(j, carry):
        pltpu.sync_copy(X.at[pl.ds(rbase + ZR * j, ZR)], segb)

        def _so(i, cc):
            sp = _rsqrt16(djall[ZR * j + i])
            for f in range(HALF // 16):
                segb[i, pl.ds(16 * f, 16)] = segb[i, pl.ds(16 * f, 16)] * sp
            return cc
        lax.fori_loop(0, ZR, _so, 0)
        pltpu.sync_copy(segb, out_hbm.at[pl.ds(obase + ZR * j, ZR)])
        return carry
    lax.fori_loop(0, nseg, _out_seg, 0)


def _sc_prop(h2, edges, cv, nch):
    cpt = nch // 16
    f = pl.kernel(
        functools.partial(_sc_prop_body, cpt),
        out_type=[jax.ShapeDtypeStruct((2 * N_PAD, HALF), jnp.float32),
                  jax.ShapeDtypeStruct((N_PAD, 16), jnp.float32)],
        mesh=plsc.VectorSubcoreMesh(**_MESH),
        compiler_params=pltpu.CompilerParams(use_tc_tiling_on_sc=False),
        scratch_types=[
            pltpu.VMEM_SHARED((N_PAD, HALF), jnp.float32),
            pltpu.VMEM_SHARED((N_PAD, HALF), jnp.float32),
            pltpu.VMEM_SHARED((N_PAD, 16), jnp.float32),
            pltpu.VMEM((2, EK), jnp.int32),
            pltpu.VMEM((2, EK), jnp.int32),
            pltpu.VMEM((EK,), jnp.int32),
            pltpu.VMEM((EK,), jnp.int32),
            pltpu.VMEM((EK, HALF), jnp.float32),
            pltpu.VMEM((EK, HALF), jnp.float32),
            pltpu.VMEM((ZR, HALF), jnp.float32),
            pltpu.VMEM((EK, 16), jnp.float32),
            pltpu.VMEM((ZR, 16), jnp.float32),
            pltpu.VMEM((RPT, 16), jnp.float32),
            pltpu.VMEM((16,), jnp.int32),
            pltpu.SemaphoreType.DMA,
            pltpu.SemaphoreType.DMA,
            pltpu.SemaphoreType.DMA,
            pltpu.SemaphoreType.DMA,
            pltpu.SemaphoreType.DMA,
            pltpu.SemaphoreType.DMA,
            pltpu.SemaphoreType.DMA,
            pltpu.SemaphoreType.DMA,
        ],
    )
    return f(h2, edges, cv)


def _sc_final_body(cpt, h_hbm, ed_hbm, dis_hbm, y_hbm,
                   X16, Y16, ebuf, ebuf1, dbuf0, dbuf1, rows16, rows16b,
                   rowbuf16, dis2b,
                   gsem0, gsem1, ssem0, ssem1, esem0, esem1):
    c = lax.axis_index("c")
    s = lax.axis_index("s")
    rbase = s * RPT
    cbase = s * cpt

    @pl.when(c == 0)
    def _():
        pltpu.sync_copy(dis_hbm.at[pl.ds(rbase, RPT)], dis2b)
        pltpu.sync_copy(h_hbm.at[pl.ds(rbase, RPT)], rowbuf16)

        def _s0(i, carry):
            rowbuf16[i] = rowbuf16[i] * dis2b[i]
            return carry
        lax.fori_loop(0, RPT, _s0, 0)
        pltpu.sync_copy(rowbuf16, X16.at[pl.ds(rbase, RPT)])
        pltpu.sync_copy(rowbuf16, Y16.at[pl.ds(rbase, RPT)])
        plsc.subcore_barrier()

        ebufs = [ebuf, ebuf1]
        dbufs = [dbuf0, dbuf1]
        rowss = [rows16, rows16b]
        gsems = [gsem0, gsem1]
        ssems = [ssem0, ssem1]
        esems = [esem0, esem1]
        npair = cpt // 2
        for b in range(2):
            pltpu.async_copy(ed_hbm.at[cbase + b], ebufs[b], esems[b])

        def _pair(g, carry):
            hs = []
            for b in range(2):
                pltpu.make_async_copy(
                    ed_hbm.at[0], ebufs[b], esems[b]).wait()

                @pl.when(g > 0)
                def _(b=b):
                    pltpu.make_async_copy(
                        dis_hbm.at[pl.ds(0, EK)], rowss[b], ssems[b]).wait()
                hs.append(pltpu.async_copy(
                    X16.at[ebufs[b].at[0]], rowss[b], gsems[b]))
            for b in range(2):
                hs[b].wait()
                for q in range(EK // 16):
                    dbufs[b][pl.ds(16 * q, 16)] = ebufs[b][1,
                                                           pl.ds(16 * q, 16)]
                pltpu.async_copy(rowss[b], Y16.at[dbufs[b]], ssems[b],
                                 add=True)

                @pl.when(g < npair - 1)
                def _(b=b, g=g):
                    pltpu.async_copy(ed_hbm.at[cbase + 2 * (g + 1) + b],
                                     ebufs[b], esems[b])
            return carry
        lax.fori_loop(0, npair, _pair, 0)
        for b in range(2):
            pltpu.make_async_copy(
                dis_hbm.at[pl.ds(0, EK)], rowss[b], ssems[b]).wait()
        plsc.subcore_barrier()

        pltpu.sync_copy(Y16.at[pl.ds(rbase, RPT)], rowbuf16)

        def _s1(i, carry):
            rowbuf16[i] = rowbuf16[i] * dis2b[i]
            return carry
        lax.fori_loop(0, RPT, _s1, 0)
        pltpu.sync_copy(rowbuf16, y_hbm.at[pl.ds(rbase, RPT)])


def _sc_final(h3, edges, dis, nch):
    cpt = nch // 16
    f = pl.kernel(
        functools.partial(_sc_final_body, cpt),
        out_type=jax.ShapeDtypeStruct((N_PAD, OUT), jnp.float32),
        mesh=plsc.VectorSubcoreMesh(**_MESH),
        compiler_params=pltpu.CompilerParams(use_tc_tiling_on_sc=False),
        scratch_types=[
            pltpu.VMEM_SHARED((N_PAD, OUT), jnp.float32),
            pltpu.VMEM_SHARED((N_PAD, OUT), jnp.float32),
            pltpu.VMEM((2, EK), jnp.int32),
            pltpu.VMEM((2, EK), jnp.int32),
            pltpu.VMEM((EK,), jnp.int32),
            pltpu.VMEM((EK,), jnp.int32),
            pltpu.VMEM((EK, OUT), jnp.float32),
            pltpu.VMEM((EK, OUT), jnp.float32),
            pltpu.VMEM((RPT, OUT), jnp.float32),
            pltpu.VMEM((RPT, 16), jnp.float32),
            pltpu.SemaphoreType.DMA,
            pltpu.SemaphoreType.DMA,
            pltpu.SemaphoreType.DMA,
            pltpu.SemaphoreType.DMA,
            pltpu.SemaphoreType.DMA,
            pltpu.SemaphoreType.DMA,
        ],
    )
    return f(h3, edges, dis)


def kernel(x, edge_index, W1, b1, W2, b2, Wc, bc, conv_time):
    N = x.shape[0]
    E = edge_index.shape[1]
    nch = -(-E // EK)
    nch = -(-nch // 32) * 32
    epad = nch * EK

    padv = jnp.full((epad - E,), N, jnp.int32)
    src = jnp.concatenate([edge_index[0].astype(jnp.int32), padv])
    dst = jnp.concatenate([edge_index[1].astype(jnp.int32), padv])
    edges = jnp.stack([src.reshape(nch, EK), dst.reshape(nch, EK)], axis=1)

    x_pad = jnp.pad(x, ((0, N_PAD - N), (0, 0)))
    cv = jnp.full((16,), conv_time, jnp.int32)

    h2 = _tc_front(x_pad, W1, b1, W2, b2)
    hp, dis = _sc_prop(h2, edges, cv, nch)
    h3 = _tc_cls(hp, Wc, bc)
    y = _sc_final(h3, edges, dis, nch)
    return y[:N]
